# baseline (device time: 106152 ns/iter reference)
import jax
import jax.numpy as jnp
from jax import lax
from jax.experimental import pallas as pl
from jax.experimental.pallas import tpu as pltpu

N_DEV = 16


def kernel(x, w_mat, scale_x, scale_w):
    m_per, k = x.shape
    _, n_per = w_mat.shape

    n_r = N_DEV // 2
    n_l = N_DEV - 1 - n_r
    n_sub = 4
    m_sub = m_per // n_sub

    def body(x_ref, w_ref, sx_ref, sw_ref, out_ref,
             gather_ref,
             r_send_sems, r_recv_sems, l_send_sems, l_recv_sems):
        my = lax.axis_index("i")
        left = lax.rem(my + N_DEV - 1, N_DEV)
        right = lax.rem(my + 1, N_DEV)

        barrier_sem = pltpu.get_barrier_semaphore()
        for nbr in (left, right):
            pl.semaphore_signal(
                barrier_sem, inc=1,
                device_id=(nbr,), device_id_type=pl.DeviceIdType.MESH,
            )
        pl.semaphore_wait(barrier_sem, 2)

        scale = sx_ref[0, 0] * sw_ref[0, 0]

        def compute_chunk(origin, chunk):
            acc = lax.dot_general(
                chunk, w_ref[...],
                (((1,), (0,)), ((), ())),
                preferred_element_type=jnp.int32,
            )
            y = acc.astype(jnp.float32) * scale
            out_ref[pl.ds(origin * m_per, m_per), :] = (
                y / (1.0 + jnp.exp(-jnp.clip(y, -60.0, 60.0)))
            )

        def compute(origin):
            compute_chunk(
                origin, gather_ref[pl.ds(origin, 1)].reshape(m_per, k)
            )

        def hop(h, s, stream_sign, nbr, send_sems, recv_sems):
            slot = lax.rem(my - stream_sign * h + 2 * N_DEV, N_DEV)
            src = x_ref.at[s] if h == 0 else gather_ref.at[slot, s]
            return pltpu.make_async_remote_copy(
                src_ref=src,
                dst_ref=gather_ref.at[slot, s],
                send_sem=send_sems.at[h, s],
                recv_sem=recv_sems.at[h, s],
                device_id=(nbr,),
                device_id_type=pl.DeviceIdType.MESH,
            )

        r_rdma = [[hop(h, s, 1, right, r_send_sems, r_recv_sems)
                   for s in range(n_sub)] for h in range(n_r)]
        l_rdma = [[hop(h, s, -1, left, l_send_sems, l_recv_sems)
                   for s in range(n_sub)] for h in range(n_l)]

        for s in range(n_sub):
            r_rdma[0][s].start()
            l_rdma[0][s].start()
        compute_chunk(my, x_ref[...].reshape(m_per, k))

        for h in range(n_r):
            for s in range(n_sub):
                r_rdma[h][s].wait_recv()
                if h + 1 < n_r:
                    r_rdma[h + 1][s].start()
                if h < n_l:
                    l_rdma[h][s].wait_recv()
                    if h + 1 < n_l:
                        l_rdma[h + 1][s].start()
            compute(lax.rem(my - 1 - h + N_DEV, N_DEV))
            if h < n_l:
                compute(lax.rem(my + 1 + h, N_DEV))

        for hops in (r_rdma, l_rdma):
            for subs in hops:
                for r in subs:
                    r.wait_send()

    return pl.pallas_call(
        body,
        out_shape=jax.ShapeDtypeStruct((N_DEV * m_per, n_per), jnp.float32),
        in_specs=[
            pl.BlockSpec(memory_space=pltpu.VMEM),
            pl.BlockSpec(memory_space=pltpu.VMEM),
            pl.BlockSpec(memory_space=pltpu.SMEM),
            pl.BlockSpec(memory_space=pltpu.SMEM),
        ],
        out_specs=pl.BlockSpec(memory_space=pltpu.VMEM),
        scratch_shapes=[
            pltpu.VMEM((N_DEV, n_sub, m_sub, k), jnp.int8),
            pltpu.SemaphoreType.DMA((n_r, n_sub)),
            pltpu.SemaphoreType.DMA((n_r, n_sub)),
            pltpu.SemaphoreType.DMA((n_l, n_sub)),
            pltpu.SemaphoreType.DMA((n_l, n_sub)),
        ],
        compiler_params=pltpu.CompilerParams(collective_id=0),
    )(x.reshape(n_sub, m_sub, k), w_mat,
      scale_x.reshape(1, 1), scale_w.reshape(1, 1))


# device time: 100521 ns/iter; 1.0560x vs baseline; 1.0560x over previous
import jax
import jax.numpy as jnp
from jax import lax
from jax.experimental import pallas as pl
from jax.experimental.pallas import tpu as pltpu

N_DEV = 16


def kernel(x, w_mat, scale_x, scale_w):
    m_per, k = x.shape
    _, n_per = w_mat.shape

    n_full = N_DEV // 2 - 1
    n_hops = n_full + 1
    n_sub = 2
    m_sub = m_per // n_sub

    def body(x_ref, w_ref, sx_ref, sw_ref, out_ref,
             gather_ref,
             r_send_sems, r_recv_sems, l_send_sems, l_recv_sems):
        my = lax.axis_index("i")
        left = lax.rem(my + N_DEV - 1, N_DEV)
        right = lax.rem(my + 1, N_DEV)

        barrier_sem = pltpu.get_barrier_semaphore()
        for nbr in (left, right):
            pl.semaphore_signal(
                barrier_sem, inc=1,
                device_id=(nbr,), device_id_type=pl.DeviceIdType.MESH,
            )
        pl.semaphore_wait(barrier_sem, 2)

        scale = sx_ref[0, 0] * sw_ref[0, 0]

        def epilogue(rows, n_rows, acc):
            y = acc.astype(jnp.float32) * scale
            out_ref[pl.ds(rows, n_rows), :] = (
                y / (1.0 + jnp.exp(-jnp.clip(y, -60.0, 60.0)))
            )

        def compute_chunk(origin, chunk):
            acc = lax.dot_general(
                chunk, w_ref[...],
                (((1,), (0,)), ((), ())),
                preferred_element_type=jnp.int32,
            )
            epilogue(origin * m_per, m_per, acc)

        def compute(origin):
            compute_chunk(
                origin, gather_ref[pl.ds(origin, 1)].reshape(m_per, k)
            )

        def compute_sub(origin, s):
            chunk = gather_ref[pl.ds(origin, 1), s].reshape(m_sub, k)
            acc = lax.dot_general(
                chunk, w_ref[...],
                (((1,), (0,)), ((), ())),
                preferred_element_type=jnp.int32,
            )
            epilogue(origin * m_per + s * m_sub, m_sub, acc)

        def hop(h, s, stream_sign, nbr, send_sems, recv_sems):
            slot = lax.rem(my - stream_sign * h + 2 * N_DEV, N_DEV)
            src = x_ref.at[s] if h == 0 else gather_ref.at[slot, s]
            return pltpu.make_async_remote_copy(
                src_ref=src,
                dst_ref=gather_ref.at[slot, s],
                send_sem=send_sems.at[h, s],
                recv_sem=recv_sems.at[h, s],
                device_id=(nbr,),
                device_id_type=pl.DeviceIdType.MESH,
            )

        r_rdma = [[hop(h, s, 1, right, r_send_sems, r_recv_sems)
                   for s in range(n_sub if h < n_hops - 1 else 1)]
                  for h in range(n_hops)]
        l_rdma = [[hop(h, s, -1, left, l_send_sems, l_recv_sems)
                   for s in (range(n_sub) if h < n_hops - 1 else (1,))]
                  for h in range(n_hops)]

        for s in range(n_sub):
            r_rdma[0][s].start()
            l_rdma[0][s].start()
        compute_chunk(my, x_ref[...].reshape(m_per, k))

        antipode = lax.rem(my + N_DEV // 2, N_DEV)
        for h in range(n_full):
            for s in range(n_sub):
                r_rdma[h][s].wait_recv()
                if h + 1 < n_full or s == 0:
                    r_rdma[h + 1][s if h + 1 < n_full else 0].start()
                l_rdma[h][s].wait_recv()
                if h + 1 < n_full or s == 1:
                    l_rdma[h + 1][s if h + 1 < n_full else 0].start()
            compute(lax.rem(my - 1 - h + N_DEV, N_DEV))
            compute(lax.rem(my + 1 + h, N_DEV))

        r_rdma[n_hops - 1][0].wait_recv()
        compute_sub(antipode, 0)
        l_rdma[n_hops - 1][0].wait_recv()
        compute_sub(antipode, 1)

        for hops in (r_rdma, l_rdma):
            for subs in hops:
                for r in subs:
                    r.wait_send()

    return pl.pallas_call(
        body,
        out_shape=jax.ShapeDtypeStruct((N_DEV * m_per, n_per), jnp.float32),
        in_specs=[
            pl.BlockSpec(memory_space=pltpu.VMEM),
            pl.BlockSpec(memory_space=pltpu.VMEM),
            pl.BlockSpec(memory_space=pltpu.SMEM),
            pl.BlockSpec(memory_space=pltpu.SMEM),
        ],
        out_specs=pl.BlockSpec(memory_space=pltpu.VMEM),
        scratch_shapes=[
            pltpu.VMEM((N_DEV, n_sub, m_sub, k), jnp.int8),
            pltpu.SemaphoreType.DMA((n_hops, n_sub)),
            pltpu.SemaphoreType.DMA((n_hops, n_sub)),
            pltpu.SemaphoreType.DMA((n_hops, n_sub)),
            pltpu.SemaphoreType.DMA((n_hops, n_sub)),
        ],
        compiler_params=pltpu.CompilerParams(collective_id=0),
    )(x.reshape(n_sub, m_sub, k), w_mat,
      scale_x.reshape(1, 1), scale_w.reshape(1, 1))
